# baseline (speedup 1.0000x reference)
import functools

import jax
import jax.numpy as jnp
from jax import lax
from jax.experimental import pallas as pl
from jax.experimental.pallas import tpu as pltpu
from jax.experimental.pallas import tpu_sc as plsc

NC = 2
NS = 16
NW = NC * NS
CH = 128
NBUF = 4
GRP = 8
MAX_NORM = 1.0
EPS = 1e-7


def _renorm_scatter(buf, tbuf, n_rows, d):
    n_slices = d // 16
    lanes = lax.iota(jnp.int32, 16)
    dpos = [(lanes + 16 * c) * CH for c in range(n_slices)]

    def group_body(gi, _):
        r0 = gi * GRP
        parts = []
        accs = []
        for j in range(GRP):
            p = [buf[r0 + j, pl.ds(16 * c, 16)] for c in range(n_slices)]
            parts.append(p)
            sv = p[0] * p[0]
            for c in range(1, n_slices):
                sv = sv + p[c] * p[c]
            for sh in (8, 4, 2, 1):
                sv = sv + sv.at[lanes ^ sh].get(mode="promise_in_bounds")
            accs.append(jnp.where(lanes == j, sv, 0.0))
        while len(accs) > 1:
            nxt = [accs[k] + accs[k + 1] for k in range(0, len(accs) - 1, 2)]
            if len(accs) % 2:
                nxt[-1] = nxt[-1] + accs[-1]
            accs = nxt
        acc = accs[0]
        iv = lax.bitcast_convert_type(acc, jnp.int32)
        iv = jnp.int32(0x5F3759DF) - (iv >> 1)
        y = lax.bitcast_convert_type(iv, jnp.float32)
        half = 0.5 * acc
        for _ in range(3):
            y = y * (1.5 - half * y * y)
        scale = jnp.where(acc > MAX_NORM * MAX_NORM, y * (1.0 - EPS * y), 1.0)
        for j in range(GRP):
            sj = scale.at[jnp.full((16,), j, jnp.int32)].get(
                mode="promise_in_bounds")
            for c in range(n_slices):
                plsc.store_scatter(tbuf, [dpos[c] + (r0 + j)],
                                   parts[j][c] * sj)
        return 0

    lax.fori_loop(0, n_rows // GRP, group_body, 0)


def kernel(input, speakers, lut_p, lut_s):
    B, L = input.shape
    V, D = lut_p.shape
    DH = D // 8
    BH = B // CH
    assert BH == NW and B == NW * CH

    idx_t = input.astype(jnp.int32).T
    spk = speakers.astype(jnp.int32)

    mesh = plsc.VectorSubcoreMesh(core_axis_name="c", subcore_axis_name="s")

    @functools.partial(
        pl.kernel,
        mesh=mesh,
        compiler_params=pltpu.CompilerParams(
            use_tc_tiling_on_sc=False, needs_layout_passes=False),
        out_type=[
            jax.ShapeDtypeStruct((L, DH, BH, 8 * CH), jnp.float32),
            jax.ShapeDtypeStruct((DH, BH, 8 * CH), jnp.float32),
        ],
        scratch_types=[
            pltpu.VMEM((L, CH), jnp.int32),
            pltpu.VMEM((CH,), jnp.int32),
            pltpu.VMEM((NBUF, CH, D), jnp.float32),
            pltpu.VMEM((NBUF, D * CH), jnp.float32),
            pltpu.VMEM((CH, D), jnp.float32),
            pltpu.VMEM((D * CH,), jnp.float32),
            pltpu.SemaphoreType.DMA((NBUF,)),
            pltpu.SemaphoreType.DMA((NBUF,)),
            pltpu.SemaphoreType.DMA,
        ],
    )
    def run(idx_hbm, spk_hbm, lut_p_hbm, lut_s_hbm, out_hbm, ident_hbm,
            idx_v, spk_v, rows_v, tp_v, srows_v, stp_v, gsem, osem, ssem):
        cid = lax.axis_index("c")
        sid = lax.axis_index("s")
        wid = sid * NC + cid
        b0 = wid * CH

        pltpu.sync_copy(idx_hbm.at[:, pl.ds(b0, CH)], idx_v)
        pltpu.sync_copy(spk_hbm.at[pl.ds(b0, CH)], spk_v)

        pltpu.async_copy(lut_s_hbm.at[spk_v], srows_v, ssem).wait()
        _renorm_scatter(srows_v, stp_v, CH, D)
        for dh in range(DH):
            pltpu.sync_copy(stp_v.at[pl.ds(dh * 8 * CH, 8 * CH)],
                            ident_hbm.at[dh, wid])

        def start_gather(l, p):
            pltpu.async_copy(
                lut_p_hbm.at[idx_v.at[l]], rows_v.at[p], gsem.at[p])

        def wait_gather(l, p):
            pltpu.make_async_copy(
                lut_p_hbm.at[idx_v.at[l]], rows_v.at[p], gsem.at[p]).wait()

        def start_write(l, p):
            for dh in range(DH):
                pltpu.async_copy(
                    tp_v.at[p, pl.ds(dh * 8 * CH, 8 * CH)],
                    out_hbm.at[l, dh, wid], osem.at[p])

        def wait_write(l, p):
            for dh in range(DH):
                pltpu.make_async_copy(
                    tp_v.at[p, pl.ds(dh * 8 * CH, 8 * CH)],
                    out_hbm.at[l, dh, wid], osem.at[p]).wait()

        def body(l, p, first, last):
            wait_gather(l, p)
            if not first:
                wait_write(l - NBUF, p)
            _renorm_scatter(rows_v.at[p], tp_v.at[p], CH, D)
            start_write(l, p)
            h = l + 2
            if not last:
                start_gather(h, (p + 2) % NBUF)

        start_gather(jnp.int32(0), 0)
        start_gather(jnp.int32(1), 1)

        for p in range(NBUF):
            body(jnp.int32(p), p, first=True, last=False)

        def round_body(i, _):
            l0 = i * NBUF
            for p in range(NBUF):
                body(l0 + p, p, first=False, last=False)
            return 0

        lax.fori_loop(1, L // NBUF - 1, round_body, 0)

        l0 = L - NBUF
        for p in range(NBUF):
            body(jnp.int32(l0 + p), p, first=False, last=(p >= 2))

        for p in range(NBUF):
            wait_write(jnp.int32(l0 + p), p)

    out5, id4 = run(idx_t, spk, lut_p, lut_s)
    out5 = out5.reshape(L, DH, BH, 8, CH)
    id4 = id4.reshape(DH, BH, 8, CH)
    out = out5.transpose(2, 4, 0, 1, 3).reshape(B, L, D)
    ident = id4.transpose(1, 3, 0, 2).reshape(B, D)
    return out, ident


# speedup vs baseline: 1.6455x; 1.6455x over previous
import functools

import jax
import jax.numpy as jnp
from jax import lax
from jax.experimental import pallas as pl
from jax.experimental.pallas import tpu as pltpu
from jax.experimental.pallas import tpu_sc as plsc

NC = 2
NS = 16
NW = NC * NS
CH = 128
PW = 129
NBUF = 4
GRP = 8
MAX_NORM = 1.0
EPS = 1e-7


def _renorm_scatter(buf, tbuf, n_rows, d):
    n_slices = d // 16
    lanes = lax.iota(jnp.int32, 16)
    dvec = [lanes + 16 * c for c in range(n_slices)]

    def group_body(gi, _):
        r0 = gi * GRP
        parts = []
        accs = []
        for j in range(GRP):
            p = [buf[r0 + j, pl.ds(16 * c, 16)] for c in range(n_slices)]
            parts.append(p)
            sv = p[0] * p[0]
            for c in range(1, n_slices):
                sv = sv + p[c] * p[c]
            for sh in (8, 4, 2, 1):
                sv = sv + sv.at[lanes ^ sh].get(mode="promise_in_bounds")
            accs.append(jnp.where(lanes == j, sv, 0.0))
        while len(accs) > 1:
            nxt = [accs[k] + accs[k + 1] for k in range(0, len(accs) - 1, 2)]
            if len(accs) % 2:
                nxt[-1] = nxt[-1] + accs[-1]
            accs = nxt
        acc = accs[0]
        iv = lax.bitcast_convert_type(acc, jnp.int32)
        iv = jnp.int32(0x5F3759DF) - (iv >> 1)
        y = lax.bitcast_convert_type(iv, jnp.float32)
        half = 0.5 * acc
        for _ in range(3):
            y = y * (1.5 - half * y * y)
        scale = jnp.where(acc > MAX_NORM * MAX_NORM, y * (1.0 - EPS * y), 1.0)
        for j in range(GRP):
            sj = scale.at[jnp.full((16,), j, jnp.int32)].get(
                mode="promise_in_bounds")
            bvec = jnp.full((16,), r0 + j, jnp.int32)
            for c in range(n_slices):
                plsc.store_scatter(tbuf, [dvec[c], bvec],
                                   parts[j][c] * sj)
        return 0

    lax.fori_loop(0, n_rows // GRP, group_body, 0)


def kernel(input, speakers, lut_p, lut_s):
    B, L = input.shape
    V, D = lut_p.shape
    DH = D // 8
    BH = B // CH
    assert BH == NW and B == NW * CH

    idx_t = input.astype(jnp.int32).T
    spk = speakers.astype(jnp.int32)

    mesh = plsc.VectorSubcoreMesh(core_axis_name="c", subcore_axis_name="s")

    @functools.partial(
        pl.kernel,
        mesh=mesh,
        compiler_params=pltpu.CompilerParams(
            use_tc_tiling_on_sc=False, needs_layout_passes=False),
        out_type=[
            jax.ShapeDtypeStruct((L, DH, BH, 8, CH), jnp.float32),
            jax.ShapeDtypeStruct((DH, BH, 8, CH), jnp.float32),
        ],
        scratch_types=[
            pltpu.VMEM((L, CH), jnp.int32),
            pltpu.VMEM((CH,), jnp.int32),
            pltpu.VMEM((NBUF, CH, D), jnp.float32),
            pltpu.VMEM((NBUF, D, PW), jnp.float32),
            pltpu.VMEM((CH, D), jnp.float32),
            pltpu.VMEM((D, PW), jnp.float32),
            pltpu.SemaphoreType.DMA((NBUF,)),
            pltpu.SemaphoreType.DMA((NBUF,)),
            pltpu.SemaphoreType.DMA,
        ],
    )
    def run(idx_hbm, spk_hbm, lut_p_hbm, lut_s_hbm, out_hbm, ident_hbm,
            idx_v, spk_v, rows_v, tp_v, srows_v, stp_v, gsem, osem, ssem):
        cid = lax.axis_index("c")
        sid = lax.axis_index("s")
        wid = sid * NC + cid
        b0 = wid * CH

        pltpu.sync_copy(idx_hbm.at[:, pl.ds(b0, CH)], idx_v)
        pltpu.sync_copy(spk_hbm.at[pl.ds(b0, CH)], spk_v)

        pltpu.async_copy(lut_s_hbm.at[spk_v], srows_v, ssem).wait()
        _renorm_scatter(srows_v, stp_v, CH, D)
        for dh in range(DH):
            pltpu.sync_copy(stp_v.at[pl.ds(dh * 8, 8), pl.ds(0, CH)],
                            ident_hbm.at[dh, wid])

        def start_gather(l, p):
            pltpu.async_copy(
                lut_p_hbm.at[idx_v.at[l]], rows_v.at[p], gsem.at[p])

        def wait_gather(l, p):
            pltpu.make_async_copy(
                lut_p_hbm.at[idx_v.at[l]], rows_v.at[p], gsem.at[p]).wait()

        def start_write(l, p):
            for dh in range(DH):
                pltpu.async_copy(
                    tp_v.at[p, pl.ds(dh * 8, 8), pl.ds(0, CH)],
                    out_hbm.at[l, dh, wid], osem.at[p])

        def wait_write(l, p):
            for dh in range(DH):
                pltpu.make_async_copy(
                    tp_v.at[p, pl.ds(dh * 8, 8), pl.ds(0, CH)],
                    out_hbm.at[l, dh, wid], osem.at[p]).wait()

        def body(l, p, first, last):
            wait_gather(l, p)
            if not first:
                wait_write(l - NBUF, p)
            _renorm_scatter(rows_v.at[p], tp_v.at[p], CH, D)
            start_write(l, p)
            h = l + 2
            if not last:
                start_gather(h, (p + 2) % NBUF)

        start_gather(jnp.int32(0), 0)
        start_gather(jnp.int32(1), 1)

        for p in range(NBUF):
            body(jnp.int32(p), p, first=True, last=False)

        def round_body(i, _):
            l0 = i * NBUF
            for p in range(NBUF):
                body(l0 + p, p, first=False, last=False)
            return 0

        lax.fori_loop(1, L // NBUF - 1, round_body, 0)

        l0 = L - NBUF
        for p in range(NBUF):
            body(jnp.int32(l0 + p), p, first=False, last=(p >= 2))

        for p in range(NBUF):
            wait_write(jnp.int32(l0 + p), p)

    out5, id4 = run(idx_t, spk, lut_p, lut_s)
    out = out5.transpose(2, 4, 0, 1, 3).reshape(B, L, D)
    ident = id4.transpose(1, 3, 0, 2).reshape(B, D)
    return out, ident


# speedup vs baseline: 1.7466x; 1.0615x over previous
import functools

import jax
import jax.numpy as jnp
from jax import lax
from jax.experimental import pallas as pl
from jax.experimental.pallas import tpu as pltpu
from jax.experimental.pallas import tpu_sc as plsc

NC = 2
NS = 16
NW = NC * NS
CH = 128
PW = 129
NBUF = 4
GRP = 8
MAX_NORM = 1.0
EPS = 1e-7


def _renorm_scatter(buf, tbuf, n_rows, d):
    n_slices = d // 16
    lanes = lax.iota(jnp.int32, 16)
    dvec = [lanes + 16 * c for c in range(n_slices)]

    def group_body(gi, _):
        r0 = gi * GRP
        parts = []
        accs = []
        for j in range(GRP):
            p = [buf[r0 + j, pl.ds(16 * c, 16)] for c in range(n_slices)]
            parts.append(p)
            sv = p[0] * p[0]
            for c in range(1, n_slices):
                sv = sv + p[c] * p[c]
            for sh in (8, 4, 2, 1):
                sv = sv + sv.at[lanes ^ sh].get(mode="promise_in_bounds")
            accs.append(jnp.where(lanes == j, sv, 0.0))
        while len(accs) > 1:
            nxt = [accs[k] + accs[k + 1] for k in range(0, len(accs) - 1, 2)]
            if len(accs) % 2:
                nxt[-1] = nxt[-1] + accs[-1]
            accs = nxt
        acc = accs[0]
        iv = lax.bitcast_convert_type(acc, jnp.int32)
        iv = jnp.int32(0x5F3759DF) - (iv >> 1)
        y = lax.bitcast_convert_type(iv, jnp.float32)
        half = 0.5 * acc
        for _ in range(3):
            y = y * (1.5 - half * y * y)
        scale = jnp.where(acc > MAX_NORM * MAX_NORM, y * (1.0 - EPS * y), 1.0)
        for j in range(GRP):
            sj = scale.at[jnp.full((16,), j, jnp.int32)].get(
                mode="promise_in_bounds")
            bvec = jnp.full((16,), r0 + j, jnp.int32)
            for c in range(n_slices):
                plsc.store_scatter(tbuf, [dvec[c], bvec],
                                   parts[j][c] * sj)
        return 0

    lax.fori_loop(0, n_rows // GRP, group_body, 0)


def kernel(input, speakers, lut_p, lut_s):
    B, L = input.shape
    V, D = lut_p.shape
    DH = D // 8
    BH = B // CH
    assert BH == NW and B == NW * CH

    idx_t = input.astype(jnp.int32).T
    spk = speakers.astype(jnp.int32)
    lut_pad = jnp.pad(lut_p, ((0, 0), (0, D)))

    mesh = plsc.VectorSubcoreMesh(core_axis_name="c", subcore_axis_name="s")

    @functools.partial(
        pl.kernel,
        mesh=mesh,
        compiler_params=pltpu.CompilerParams(
            use_tc_tiling_on_sc=False, needs_layout_passes=False),
        out_type=[
            jax.ShapeDtypeStruct((L, DH, BH, 8, CH), jnp.float32),
            jax.ShapeDtypeStruct((DH, BH, 8, CH), jnp.float32),
        ],
        scratch_types=[
            pltpu.VMEM((L, CH), jnp.int32),
            pltpu.VMEM((CH,), jnp.int32),
            pltpu.VMEM((NBUF, CH, 2 * D), jnp.float32),
            pltpu.VMEM((2, D, PW), jnp.float32),
            pltpu.VMEM((CH, D), jnp.float32),
            pltpu.VMEM((D, PW), jnp.float32),
            pltpu.SemaphoreType.DMA((NBUF,)),
            pltpu.SemaphoreType.DMA((NBUF,)),
            pltpu.SemaphoreType.DMA,
        ],
    )
    def run(idx_hbm, spk_hbm, lut_p_hbm, lut_s_hbm, out_hbm, ident_hbm,
            idx_v, spk_v, rows_v, tp_v, srows_v, stp_v, gsem, osem, ssem):
        cid = lax.axis_index("c")
        sid = lax.axis_index("s")
        wid = sid * NC + cid
        b0 = wid * CH

        pltpu.sync_copy(idx_hbm.at[:, pl.ds(b0, CH)], idx_v)
        pltpu.sync_copy(spk_hbm.at[pl.ds(b0, CH)], spk_v)

        pltpu.async_copy(lut_s_hbm.at[spk_v], srows_v, ssem).wait()
        _renorm_scatter(srows_v, stp_v, CH, D)
        for dh in range(DH):
            pltpu.sync_copy(stp_v.at[pl.ds(dh * 8, 8), pl.ds(0, CH)],
                            ident_hbm.at[dh, wid])

        def start_gather(l, p):
            pltpu.async_copy(
                lut_p_hbm.at[idx_v.at[l]], rows_v.at[p], gsem.at[p])

        def wait_gather(l, p):
            pltpu.make_async_copy(
                lut_p_hbm.at[idx_v.at[l]], rows_v.at[p], gsem.at[p]).wait()

        def start_write(l, pp):
            for dh in range(DH):
                pltpu.async_copy(
                    tp_v.at[pp, pl.ds(dh * 8, 8), pl.ds(0, CH)],
                    out_hbm.at[l, dh, wid], osem.at[pp])

        def wait_write(l, pp):
            for dh in range(DH):
                pltpu.make_async_copy(
                    tp_v.at[pp, pl.ds(dh * 8, 8), pl.ds(0, CH)],
                    out_hbm.at[l, dh, wid], osem.at[pp]).wait()

        def body(l, p, first, last):
            pp = p % 2
            wait_gather(l, p)
            if not first:
                wait_write(l - 2, pp)
            _renorm_scatter(rows_v.at[p], tp_v.at[pp], CH, D)
            start_write(l, pp)
            h = l + 2
            if not last:
                start_gather(h, (p + 2) % NBUF)

        start_gather(jnp.int32(0), 0)
        start_gather(jnp.int32(1), 1)

        for p in range(NBUF):
            body(jnp.int32(p), p, first=(p < 2), last=False)

        def round_body(i, _):
            l0 = i * NBUF
            for p in range(NBUF):
                body(l0 + p, p, first=False, last=False)
            return 0

        lax.fori_loop(1, L // NBUF - 1, round_body, 0)

        l0 = L - NBUF
        for p in range(NBUF):
            body(jnp.int32(l0 + p), p, first=False, last=(p >= 2))

        for p in (2, 3):
            wait_write(jnp.int32(l0 + p), p % 2)

    out5, id4 = run(idx_t, spk, lut_pad, lut_s)
    out = out5.transpose(2, 4, 0, 1, 3).reshape(B, L, D)
    ident = id4.transpose(1, 3, 0, 2).reshape(B, D)
    return out, ident


# speedup vs baseline: 1.7840x; 1.0214x over previous
import functools

import jax
import jax.numpy as jnp
from jax import lax
from jax.experimental import pallas as pl
from jax.experimental.pallas import tpu as pltpu
from jax.experimental.pallas import tpu_sc as plsc

NC = 2
NS = 16
NW = NC * NS
CH = 128
PW = 129
NBUF = 4
GRP = 8
MAX_NORM = 1.0
EPS = 1e-7


def _renorm_scatter(buf, tbuf, n_rows, d):
    n_slices = d // 16
    lanes = lax.iota(jnp.int32, 16)
    dhvec = [(lanes + 16 * c) >> 3 for c in range(n_slices)]
    dlvec = [(lanes + 16 * c) & 7 for c in range(n_slices)]

    def group_body(gi):
        r0 = gi * GRP
        parts = []
        accs = []
        for j in range(GRP):
            p = [buf[r0 + j, pl.ds(16 * c, 16)] for c in range(n_slices)]
            parts.append(p)
            sv = p[0] * p[0]
            for c in range(1, n_slices):
                sv = sv + p[c] * p[c]
            for sh in (8, 4, 2, 1):
                sv = sv + sv.at[lanes ^ sh].get(mode="promise_in_bounds")
            accs.append(jnp.where(lanes == j, sv, 0.0))
        while len(accs) > 1:
            nxt = [accs[k] + accs[k + 1] for k in range(0, len(accs) - 1, 2)]
            if len(accs) % 2:
                nxt[-1] = nxt[-1] + accs[-1]
            accs = nxt
        acc = accs[0]
        iv = lax.bitcast_convert_type(acc, jnp.int32)
        iv = jnp.int32(0x5F3759DF) - (iv >> 1)
        y = lax.bitcast_convert_type(iv, jnp.float32)
        half = 0.5 * acc
        for _ in range(3):
            y = y * (1.5 - half * y * y)
        scale = jnp.where(acc > MAX_NORM * MAX_NORM, y * (1.0 - EPS * y), 1.0)
        for j in range(GRP):
            sj = scale.at[jnp.full((16,), j, jnp.int32)].get(
                mode="promise_in_bounds")
            bvec = jnp.full((16,), r0 + j, jnp.int32)
            for c in range(n_slices):
                plsc.store_scatter(tbuf, [dhvec[c], dlvec[c], bvec],
                                   parts[j][c] * sj)

    def pair_body(gi, _):
        group_body(2 * gi)
        group_body(2 * gi + 1)
        return 0

    lax.fori_loop(0, n_rows // (2 * GRP), pair_body, 0)


def kernel(input, speakers, lut_p, lut_s):
    B, L = input.shape
    V, D = lut_p.shape
    DH = D // 8
    BH = B // CH
    assert BH == NW and B == NW * CH

    idx_t = input.astype(jnp.int32).T
    spk = speakers.astype(jnp.int32)
    lut_pad = jnp.pad(lut_p, ((0, 0), (0, D)))

    mesh = plsc.VectorSubcoreMesh(core_axis_name="c", subcore_axis_name="s")

    @functools.partial(
        pl.kernel,
        mesh=mesh,
        compiler_params=pltpu.CompilerParams(
            use_tc_tiling_on_sc=False, needs_layout_passes=False),
        out_type=[
            jax.ShapeDtypeStruct((L, DH, BH, 8, CH), jnp.float32),
            jax.ShapeDtypeStruct((DH, BH, 8, CH), jnp.float32),
        ],
        scratch_types=[
            pltpu.VMEM((L, CH), jnp.int32),
            pltpu.VMEM((CH,), jnp.int32),
            pltpu.VMEM((NBUF, CH, 2 * D), jnp.float32),
            pltpu.VMEM((2, DH, 8, PW), jnp.float32),
            pltpu.VMEM((CH, D), jnp.float32),
            pltpu.VMEM((DH, 8, PW), jnp.float32),
            pltpu.SemaphoreType.DMA((NBUF,)),
            pltpu.SemaphoreType.DMA((NBUF,)),
            pltpu.SemaphoreType.DMA,
        ],
    )
    def run(idx_hbm, spk_hbm, lut_p_hbm, lut_s_hbm, out_hbm, ident_hbm,
            idx_v, spk_v, rows_v, tp_v, srows_v, stp_v, gsem, osem, ssem):
        cid = lax.axis_index("c")
        sid = lax.axis_index("s")
        wid = sid * NC + cid
        b0 = wid * CH

        pltpu.sync_copy(idx_hbm.at[:, pl.ds(b0, CH)], idx_v)
        pltpu.sync_copy(spk_hbm.at[pl.ds(b0, CH)], spk_v)

        pltpu.async_copy(lut_s_hbm.at[spk_v], srows_v, ssem).wait()
        _renorm_scatter(srows_v, stp_v, CH, D)
        pltpu.sync_copy(stp_v.at[:, :, pl.ds(0, CH)],
                        ident_hbm.at[:, wid])

        def start_gather(l, p):
            pltpu.async_copy(
                lut_p_hbm.at[idx_v.at[l]], rows_v.at[p], gsem.at[p])

        def wait_gather(l, p):
            pltpu.make_async_copy(
                lut_p_hbm.at[idx_v.at[l]], rows_v.at[p], gsem.at[p]).wait()

        def start_write(l, pp):
            pltpu.async_copy(tp_v.at[pp, :, :, pl.ds(0, CH)],
                             out_hbm.at[l, :, wid], osem.at[pp])

        def wait_write(l, pp):
            pltpu.make_async_copy(tp_v.at[pp, :, :, pl.ds(0, CH)],
                                  out_hbm.at[l, :, wid], osem.at[pp]).wait()

        def body(l, p, first, last):
            pp = p % 2
            wait_gather(l, p)
            if not first:
                wait_write(l - 2, pp)
            _renorm_scatter(rows_v.at[p], tp_v.at[pp], CH, D)
            start_write(l, pp)
            h = l + 2
            if not last:
                start_gather(h, (p + 2) % NBUF)

        start_gather(jnp.int32(0), 0)
        start_gather(jnp.int32(1), 1)

        for p in range(NBUF):
            body(jnp.int32(p), p, first=(p < 2), last=False)

        def round_body(i, _):
            l0 = i * NBUF
            for p in range(NBUF):
                body(l0 + p, p, first=False, last=False)
            return 0

        lax.fori_loop(1, L // NBUF - 1, round_body, 0)

        l0 = L - NBUF
        for p in range(NBUF):
            body(jnp.int32(l0 + p), p, first=False, last=(p >= 2))

        for p in (2, 3):
            wait_write(jnp.int32(l0 + p), p % 2)

    out5, id4 = run(idx_t, spk, lut_pad, lut_s)
    out = out5.transpose(2, 4, 0, 1, 3).reshape(B, L, D)
    ident = id4.transpose(1, 3, 0, 2).reshape(B, D)
    return out, ident
